# baseline (speedup 1.0000x reference)
import functools
import jax
import jax.numpy as jnp
from jax import lax
from jax.experimental import pallas as pl
from jax.experimental.pallas import tpu as pltpu
from jax.experimental.pallas import tpu_sc as plsc

D = 64
NC = 2
NS = 16
NW = NC * NS
K = 4
C = K * 128


def _emb_body(idx_hbm, table_hbm, out_hbm, idx_v, rows_v, gsem):
    wid = lax.axis_index("s") * NC + lax.axis_index("c")
    b_per_w = out_hbm.shape[0] // NW
    nchunk = b_per_w // C
    idx_rows_per_w = b_per_w // 128

    def chunk(ci, carry):
        rb = wid * idx_rows_per_w + ci * K
        base = wid * b_per_w + ci * C
        pltpu.sync_copy(idx_hbm.at[pl.ds(rb, K)], idx_v)
        cps = []
        for j in range(K):
            cps.append(
                pltpu.async_copy(
                    table_hbm.at[idx_v.at[j]],
                    rows_v.at[pl.ds(j * 128, 128)],
                    gsem,
                )
            )
        for cp in cps:
            cp.wait()
        pltpu.sync_copy(rows_v, out_hbm.at[pl.ds(base, C)])
        return carry

    lax.fori_loop(0, nchunk, chunk, 0)


def kernel(input, weight):
    B = input.shape[0] * input.shape[1]
    idx = input.reshape(B // 128, 128).astype(jnp.int32)

    gather = functools.partial(
        pl.kernel,
        mesh=plsc.VectorSubcoreMesh(core_axis_name="c", subcore_axis_name="s"),
        out_type=jax.ShapeDtypeStruct((B, D), jnp.float32),
        scratch_types=[
            pltpu.VMEM((K, 128), jnp.int32),
            pltpu.VMEM((C, D), jnp.float32),
            pltpu.SemaphoreType.DMA,
        ],
        compiler_params=pltpu.CompilerParams(use_tc_tiling_on_sc=False),
    )(_emb_body)

    out = gather(idx, weight)
    return out.reshape(input.shape[0], input.shape[1], D)


# speedup vs baseline: 1.0247x; 1.0247x over previous
import functools
import jax
import jax.numpy as jnp
from jax import lax
from jax.experimental import pallas as pl
from jax.experimental.pallas import tpu as pltpu
from jax.experimental.pallas import tpu_sc as plsc

D = 64
NC = 2
NS = 16
NW = NC * NS
K = 4
C = K * 128
NBUF = 2


def _emb_body(idx_hbm, table_hbm, out_hbm, idx_v, rows_v, gsem0, gsem1,
              osem0, osem1):
    wid = lax.axis_index("s") * NC + lax.axis_index("c")
    b_per_w = out_hbm.shape[0] // NW
    nchunk = b_per_w // C
    idx_rows = b_per_w // 128
    gsems = [gsem0, gsem1]
    osems = [osem0, osem1]

    pltpu.sync_copy(idx_hbm.at[pl.ds(wid * idx_rows, idx_rows)], idx_v)

    def gather_cp(ci, b, j):
        return pltpu.make_async_copy(
            table_hbm.at[idx_v.at[ci * K + j]],
            rows_v.at[b].at[pl.ds(j * 128, 128)],
            gsems[b],
        )

    def out_cp(ci, b):
        return pltpu.make_async_copy(
            rows_v.at[b],
            out_hbm.at[pl.ds(wid * b_per_w + ci * C, C)],
            osems[b],
        )

    def fire_gather(ci, b):
        for j in range(K):
            gather_cp(ci, b, j).start()

    def wait_gather(ci, b):
        for j in range(K):
            gather_cp(ci, b, j).wait()

    fire_gather(0, 0)
    fire_gather(1, 1)

    def step(g, carry):
        for b in range(NBUF):
            ci = NBUF * g + b
            wait_gather(ci, b)
            out_cp(ci, b).start()
        for b in range(NBUF):
            ci = NBUF * g + b
            out_cp(ci, b).wait()

            @pl.when(g < nchunk // NBUF - 1)
            def _():
                fire_gather(ci + NBUF, b)

        return carry

    lax.fori_loop(0, nchunk // NBUF, step, 0)


def kernel(input, weight):
    B = input.shape[0] * input.shape[1]
    idx = input.reshape(B // 128, 128).astype(jnp.int32)

    gather = functools.partial(
        pl.kernel,
        mesh=plsc.VectorSubcoreMesh(core_axis_name="c", subcore_axis_name="s"),
        out_type=jax.ShapeDtypeStruct((B, D), jnp.float32),
        scratch_types=[
            pltpu.VMEM((B // 128 // NW, 128), jnp.int32),
            pltpu.VMEM((NBUF, C, D), jnp.float32),
            pltpu.SemaphoreType.DMA,
            pltpu.SemaphoreType.DMA,
            pltpu.SemaphoreType.DMA,
            pltpu.SemaphoreType.DMA,
        ],
        compiler_params=pltpu.CompilerParams(use_tc_tiling_on_sc=False),
    )(_emb_body)

    out = gather(idx, weight)
    return out.reshape(input.shape[0], input.shape[1], D)
